# Initial kernel scaffold; baseline (speedup 1.0000x reference)
import jax
import jax.numpy as jnp
from jax.experimental import pallas as pl


def kernel(enc, embed):
    raise NotImplementedError("write your pallas kernel here")


# speedup vs baseline: 1.4692x; 1.4692x over previous
import jax
import jax.numpy as jnp
from jax.experimental import pallas as pl
from jax.experimental.pallas import tpu as pltpu

_K = 512
_D = 64
_T = 2048


def _body(x_ref, emb_ref, out_ref, idx_ref, loss_ref):
    x = x_ref[...]
    emb = emb_ref[...]
    dot = jax.lax.dot_general(
        x, emb, (((1,), (1,)), ((), ())),
        preferred_element_type=jnp.float32)
    q2 = jnp.sum(x * x, axis=1, keepdims=True)
    e2 = jnp.sum(emb * emb, axis=1)
    d2 = (q2 + e2[None, :]) - 2.0 * dot
    iota = jax.lax.broadcasted_iota(jnp.int32, (_T, _K), 1)
    m = jnp.min(d2, axis=1, keepdims=True)
    closest = jnp.min(jnp.where(d2 == m, iota, _K), axis=1)
    onehot = (iota == closest[:, None]).astype(jnp.float32)
    quant = jax.lax.dot_general(
        onehot, emb, (((1,), (0,)), ((), ())),
        preferred_element_type=jnp.float32,
        precision=jax.lax.Precision.HIGHEST)
    out_ref[...] = x + (quant - x)
    idx_ref[...] = closest.reshape(1, 1, _T)
    diff = quant - x

    @pl.when(pl.program_id(0) == 0)
    def _():
        loss_ref[0, 0] = 0.0

    loss_ref[0, 0] += jnp.sum(diff * diff)


def kernel(enc, embed):
    B, C, H, W = enc.shape
    Kc, Dc = embed.shape
    N = B * C * H * W // Dc
    x = enc.reshape(N, Dc)
    nb = N // _T
    out, idx3, loss_sum = pl.pallas_call(
        _body,
        grid=(nb,),
        in_specs=[
            pl.BlockSpec((_T, Dc), lambda i: (i, 0)),
            pl.BlockSpec((Kc, Dc), lambda i: (0, 0)),
        ],
        out_specs=[
            pl.BlockSpec((_T, Dc), lambda i: (i, 0)),
            pl.BlockSpec((1, 1, _T), lambda i: (i, 0, 0)),
            pl.BlockSpec((1, 1), lambda i: (0, 0), memory_space=pltpu.SMEM),
        ],
        out_shape=[
            jax.ShapeDtypeStruct((N, Dc), jnp.float32),
            jax.ShapeDtypeStruct((nb, 1, _T), jnp.int32),
            jax.ShapeDtypeStruct((1, 1), jnp.float32),
        ],
    )(x, embed)
    mse = loss_sum[0, 0] / jnp.float32(N * Dc)
    quantize_loss = mse + mse
    quant_out = out.reshape(B, C, H, W)
    closest = idx3.reshape(B, N // B)
    return (quant_out, quantize_loss, closest)
